# baseline (speedup 1.0000x reference)
import jax
import jax.numpy as jnp
from jax.experimental import pallas as pl

K = 1024
D = 64
COMMITMENT_COST = 0.25

BN = 512


def _vq_block(z_ref, e_ref, zsq_ref, esq_ref, q_ref, idx_ref, loss_ref):
    i = pl.program_id(0)
    z = z_ref[...]
    e = e_ref[...]
    prod = jax.lax.dot_general(
        z, e, (((1,), (1,)), ((), ())),
        preferred_element_type=jnp.float32,
    )
    d = zsq_ref[...] + esq_ref[...] - 2.0 * prod
    minval = jnp.min(d, axis=1, keepdims=True)
    lanes = jax.lax.broadcasted_iota(jnp.int32, (BN, K), 1)
    idx = jnp.min(jnp.where(d == minval, lanes, K), axis=1, keepdims=True)
    idx_ref[...] = idx
    onehot = (lanes == idx).astype(jnp.float32)
    q_ref[...] = jax.lax.dot_general(
        onehot, e, (((1,), (0,)), ((), ())),
        preferred_element_type=jnp.float32,
    )
    part = (jnp.sum(minval) * ((1.0 + COMMITMENT_COST) / 16.0)).reshape(1, 1)

    @pl.when(i == 0)
    def _init():
        loss_ref[...] = part

    @pl.when(i > 0)
    def _acc():
        loss_ref[...] += part


@jax.jit
def kernel(z, embeddings):
    B, Dc, H, W = z.shape
    N = B * H * W
    z_flat = jnp.transpose(z, (0, 2, 3, 1)).reshape(N, Dc)
    zsq = jnp.sum(z_flat ** 2, axis=1, keepdims=True)
    esq = jnp.sum(embeddings ** 2, axis=1)[None, :]
    grid = N // BN
    q, idx, loss = pl.pallas_call(
        _vq_block,
        grid=(grid,),
        in_specs=[
            pl.BlockSpec((BN, Dc), lambda i: (i, 0)),
            pl.BlockSpec((K, Dc), lambda i: (0, 0)),
            pl.BlockSpec((BN, 1), lambda i: (i, 0)),
            pl.BlockSpec((1, K), lambda i: (0, 0)),
        ],
        out_specs=[
            pl.BlockSpec((BN, Dc), lambda i: (i, 0)),
            pl.BlockSpec((BN, 1), lambda i: (i, 0)),
            pl.BlockSpec((1, 1), lambda i: (0, 0)),
        ],
        out_shape=[
            jax.ShapeDtypeStruct((N, Dc), jnp.float32),
            jax.ShapeDtypeStruct((N, 1), jnp.int32),
            jax.ShapeDtypeStruct((1, 1), jnp.float32),
        ],
    )(z_flat, embeddings, zsq, esq)
    quantized = jnp.transpose(q.reshape(B, H, W, Dc), (0, 3, 1, 2))
    encoding_indices = idx.reshape(B, H, W)
    return (quantized, loss[0, 0], encoding_indices)
